# Initial kernel scaffold; baseline (speedup 1.0000x reference)
import jax
import jax.numpy as jnp
from jax.experimental import pallas as pl


def kernel(x, edge_idx, W1, a1s, a1d, W2, a2s, a2d, W3, a3s, a3d):
    raise NotImplementedError("write your pallas kernel here")


# speedup vs baseline: 9.5414x; 9.5414x over previous
import functools

import jax
import jax.numpy as jnp
from jax import lax
from jax.experimental import pallas as pl
from jax.experimental.pallas import tpu as pltpu
from jax.experimental.pallas import tpu_sc as plsc

_N = 10000
_NPAD = 10240
_E = 320000
_NB = 1000
_NSUB = 16
_EB0 = _E // _NSUB


def _mm_body(x_ref, w_ref, av_ref, sc_ref, h_ref, al_ref, *, apply_act, nk):
    k = pl.program_id(2)
    x = x_ref[0]
    if apply_act:
        x = jnp.maximum(x * sc_ref[...], 0.0)
    part = jnp.dot(x, w_ref[0, 0], preferred_element_type=jnp.float32)
    if nk == 1:
        h_ref[0] = part
    else:
        @pl.when(k == 0)
        def _():
            h_ref[0] = part

        @pl.when(k != 0)
        def _():
            h_ref[0] = h_ref[0] + part

    alp = jnp.dot(part, av_ref[0], preferred_element_type=jnp.float32)
    j = pl.program_id(1)

    @pl.when((k == 0) & (j == 0))
    def _():
        al_ref[...] = alp

    @pl.when((k != 0) | (j != 0))
    def _():
        al_ref[...] = al_ref[...] + alp


def _dense_layer(xs, W, a_s, a_d, scale, apply_act):
    nk, n, kc = xs.shape
    dout = W.shape[1]
    dc = 48 if dout == 48 else 64
    nj = dout // dc
    av = jnp.stack([a_s, a_d], axis=1)
    Wr = W.reshape(nk, kc, nj, dc).transpose(0, 2, 1, 3)
    avr = av.reshape(nj, dc, 2)
    hs, al = pl.pallas_call(
        functools.partial(_mm_body, apply_act=apply_act, nk=nk),
        grid=(n // _NB, nj, nk),
        in_specs=[
            pl.BlockSpec((1, _NB, kc), lambda i, j, k: (k, i, 0)),
            pl.BlockSpec((1, 1, kc, dc), lambda i, j, k: (k, j, 0, 0)),
            pl.BlockSpec((1, dc, 2), lambda i, j, k: (j, 0, 0)),
            pl.BlockSpec((_NB, 1), lambda i, j, k: (i, 0)),
        ],
        out_specs=[
            pl.BlockSpec((1, _NB, dc), lambda i, j, k: (j, i, 0)),
            pl.BlockSpec((_NB, 2), lambda i, j, k: (i, 0)),
        ],
        out_shape=[
            jax.ShapeDtypeStruct((nj, n, dc), jnp.float32),
            jax.ShapeDtypeStruct((n, 2), jnp.float32),
        ],
    )(xs, Wr, avr, scale)
    return hs, al[:, 0], al[:, 1]


def _zero_vec(ref, rows, width):
    z = jnp.zeros((16,), jnp.float32)

    def body(i, _):
        r = i // (width // 16)
        cc = i % (width // 16)
        ref[r, pl.ds(cc * 16, 16)] = z
        return 0

    lax.fori_loop(0, rows * (width // 16), body, 0)


def _zero_vec1d(ref, size):
    z = jnp.zeros((16,), jnp.float32)

    def body(i, _):
        ref[pl.ds(i * 16, 16)] = z
        return 0

    lax.fori_loop(0, size // 16, body, 0)


def _table_max(ref, n):
    def body(i, acc):
        return jnp.maximum(acc, ref[pl.ds(i * 16, 16)])

    acc = lax.fori_loop(0, n // 16, body, jnp.full((16,), -1e30, jnp.float32))
    m = acc[0]
    for j in range(1, 16):
        m = jnp.maximum(m, acc[j])
    return m


_S = 2000
_K = 80
_NBS = _S // _K


def _phase_b_seg(src_s, dst_s, p_s, als_v, ald_v, M):
    def body(i, _):
        q = i // (_K // 16)
        m = i % (_K // 16)
        sv = src_s[pl.ds(i * 16, 16)]
        dv = dst_s[q, pl.ds(m * 16, 16)]
        av = plsc.load_gather(als_v, [sv])
        bv = plsc.load_gather(ald_v, [dv])
        ev = av + bv
        ev = jnp.where(ev >= 0.0, ev, ev * 0.2)
        p_s[q, pl.ds(m * 16, 16)] = jnp.exp(ev - M)
        return 0

    lax.fori_loop(0, _S // 16, body, 0)


def _scatter_s(p_s, dst_s, s_sh, sem, drain_src):
    def body(r, _):
        pltpu.async_copy(p_s.at[r], s_sh.at[dst_s.at[r]], sem, add=True)
        return 0

    lax.fori_loop(0, _NBS, body, 0)
    pltpu.make_async_copy(drain_src, dst_s, sem).wait()


def _zero_acc(acc, zbuf, t):
    def zb(j, _):
        pltpu.sync_copy(zbuf, acc.at[pl.ds(t * 640 + j * 80, 80)])
        return 0

    @pl.when(t < 15)
    def _():
        lax.fori_loop(0, 8, zb, 0)

    @pl.when(t == 15)
    def _():
        lax.fori_loop(0, 5, zb, 0)


def _flush_acc(acc, out_slab_hbm, t):
    @pl.when(t < 15)
    def _():
        pltpu.sync_copy(acc.at[pl.ds(t * 640, 640)],
                        out_slab_hbm.at[pl.ds(t * 640, 640)])

    @pl.when(t == 15)
    def _():
        pltpu.sync_copy(acc.at[pl.ds(9600, 400)],
                        out_slab_hbm.at[pl.ds(9600, 400)])


def _phase_c_seg(slab, src_s, dst_s, p_s, hsf_hbm, acc,
                 ib0, ib1, buf0, buf1, g0, g1, s0, s1, Dc):
    kc = _K // 16
    ncc = Dc // 16
    nb = _NBS

    def stage(b, ib, sem, buf):
        off = slab * _N

        def ibody(i, _):
            ib[pl.ds(i * 16, 16)] = src_s[pl.ds(b * _K + i * 16, 16)] + off
            return 0

        lax.fori_loop(0, kc, ibody, 0)
        pltpu.async_copy(hsf_hbm.at[ib], buf, sem)

    def wait_dma(buf, sem):
        pltpu.make_async_copy(hsf_hbm.at[pl.ds(0, _K)], buf, sem).wait()

    def multiply(b, buf):
        def mbody(g, _):
            pvec = p_s[b, pl.ds(g * 16, 16)]
            for j in range(16):
                pe = pvec[j]
                r = g * 16 + j
                for cc in range(ncc):
                    sl = pl.ds(cc * 16, 16)
                    buf[r, sl] = buf[r, sl] * pe
            return 0

        lax.fori_loop(0, _K // 16, mbody, 0)

    stage(0, ib0, g0, buf0)

    def pair(pr, _):
        b0 = 2 * pr
        wait_dma(buf0, g0)

        @pl.when(pr > 0)
        def _():
            wait_dma(buf1, s1)

        stage(b0 + 1, ib1, g1, buf1)
        multiply(b0, buf0)
        pltpu.async_copy(buf0, acc.at[dst_s.at[b0]], s0, add=True)
        wait_dma(buf1, g1)
        wait_dma(buf0, s0)

        @pl.when(b0 + 2 < nb)
        def _():
            stage(b0 + 2, ib0, g0, buf0)

        multiply(b0 + 1, buf1)
        pltpu.async_copy(buf1, acc.at[dst_s.at[b0 + 1]], s1, add=True)
        return 0

    lax.fori_loop(0, nb // 2, pair, 0)
    if nb % 2 == 1:
        wait_dma(buf0, g0)
        multiply(nb - 1, buf0)
        pltpu.async_copy(buf0, acc.at[dst_s.at[nb - 1]], s0, add=True)
        wait_dma(buf0, s0)
    wait_dma(buf1, s1)


def _sc_body(src_hbm, dst4_hbm, als_hbm, ald_hbm, hsf_hbm, out_hbm, s_hbm,
             src_s, dst_s, p_s, als_v, ald_v, ib0, ib1, buf0, buf1, zs,
             acc, s_sh, g0, g1, s0, s1, *, Dc, spc, nseg, layer3):
    c = lax.axis_index("c")
    t = lax.axis_index("s")
    grp = t * 2 + c if layer3 else t
    ebase = grp * (nseg * _S)
    pltpu.sync_copy(als_hbm, als_v)
    pltpu.sync_copy(ald_hbm, ald_v)
    _zero_vec1d(zs, 640)
    pltpu.sync_copy(zs, s_sh.at[pl.ds(t * 640, 640)])
    plsc.subcore_barrier()
    M = jnp.maximum(_table_max(als_v, _N) + _table_max(ald_v, _N), 0.0)

    for sp in range(spc):
        slab = 0 if layer3 else c * spc + sp
        _zero_vec(buf0, 80, Dc)
        _zero_acc(acc, buf0, t)
        plsc.subcore_barrier()

        def seg_body(seg, _, sp=sp):
            e0 = ebase + seg * _S
            pltpu.sync_copy(src_hbm.at[pl.ds(e0, _S)], src_s)
            pltpu.sync_copy(dst4_hbm.at[grp].at[seg], dst_s)
            _phase_b_seg(src_s, dst_s, p_s, als_v, ald_v, M)
            if sp == 0:
                _scatter_s(p_s, dst_s, s_sh, s0, dst4_hbm.at[grp].at[seg])
            _phase_c_seg(slab, src_s, dst_s, p_s, hsf_hbm, acc,
                         ib0, ib1, buf0, buf1, g0, g1, s0, s1, Dc)
            return 0

        lax.fori_loop(0, nseg, seg_body, 0)
        plsc.subcore_barrier()
        _flush_acc(acc, out_hbm.at[c] if layer3 else out_hbm.at[slab], t)

    if layer3:
        pltpu.sync_copy(s_sh.at[pl.ds(t * 640, 640)],
                        s_hbm.at[c].at[pl.ds(t * 640, 640)])
    else:
        @pl.when(c == 0)
        def _():
            pltpu.sync_copy(s_sh.at[pl.ds(t * 640, 640)],
                            s_hbm.at[pl.ds(t * 640, 640)])


def _sc_edge_layer(src, dst, als, ald, hs, *, layer3):
    nj, n, dc = hs.shape
    hsf = hs.reshape(nj * n, dc)
    mesh = plsc.VectorSubcoreMesh(core_axis_name="c", subcore_axis_name="s")
    if layer3:
        ngrp, nseg, spc = 32, 5, 1
        out_shape = jax.ShapeDtypeStruct((2, _N, dc), jnp.float32)
        s_shape = jax.ShapeDtypeStruct((2, _NPAD), jnp.float32)
    else:
        ngrp, nseg, spc = 16, 10, nj // 2
        out_shape = jax.ShapeDtypeStruct((nj, _N, dc), jnp.float32)
        s_shape = jax.ShapeDtypeStruct((_NPAD,), jnp.float32)
    body = functools.partial(_sc_body, Dc=dc, spc=spc, nseg=nseg,
                             layer3=layer3)
    dst4 = dst.reshape(ngrp, nseg, _NBS, _K)
    kfn = pl.kernel(
        body,
        out_type=[out_shape, s_shape],
        mesh=mesh,
        compiler_params=pltpu.CompilerParams(needs_layout_passes=False,
                                             use_tc_tiling_on_sc=False),
        scratch_types=[
            pltpu.VMEM((_S,), jnp.int32),
            pltpu.VMEM((_NBS, _K), jnp.int32),
            pltpu.VMEM((_NBS, _K), jnp.float32),
            pltpu.VMEM((_N,), jnp.float32),
            pltpu.VMEM((_N,), jnp.float32),
            pltpu.VMEM((_K,), jnp.int32),
            pltpu.VMEM((_K,), jnp.int32),
            pltpu.VMEM((_K, dc), jnp.float32),
            pltpu.VMEM((_K, dc), jnp.float32),
            pltpu.VMEM((640,), jnp.float32),
            pltpu.VMEM_SHARED((_N, dc), jnp.float32),
            pltpu.VMEM_SHARED((_NPAD,), jnp.float32),
            pltpu.SemaphoreType.DMA,
            pltpu.SemaphoreType.DMA,
            pltpu.SemaphoreType.DMA,
            pltpu.SemaphoreType.DMA,
        ],
    )
    return kfn(src, dst4, als, ald, hsf)


def _softmax_body(xa_ref, xb_ref, r_ref, o_ref):
    x = (xa_ref[0] + xb_ref[0]) * r_ref[...]
    col = lax.broadcasted_iota(jnp.int32, x.shape, 1)
    x = jnp.where(col < 40, x, -1e30)
    m = jnp.max(x, axis=1, keepdims=True)
    p = jnp.exp(x - m)
    o_ref[...] = (p / jnp.sum(p, axis=1, keepdims=True))[:, :40]


def _softmax(out3, r3):
    return pl.pallas_call(
        _softmax_body,
        grid=(_N // _NB,),
        in_specs=[
            pl.BlockSpec((1, _NB, 48), lambda i: (0, i, 0)),
            pl.BlockSpec((1, _NB, 48), lambda i: (1, i, 0)),
            pl.BlockSpec((_NB, 1), lambda i: (i, 0)),
        ],
        out_specs=pl.BlockSpec((_NB, 40), lambda i: (i, 0)),
        out_shape=jax.ShapeDtypeStruct((_N, 40), jnp.float32),
    )(out3, out3, r3)


def kernel(x, edge_idx, W1, a1s, a1d, W2, a2s, a2d, W3, a3s, a3d):
    src = edge_idx[0].astype(jnp.int32)
    dst = edge_idx[1].astype(jnp.int32)
    ones = jnp.ones((_N, 1), jnp.float32)

    hs1, als1, ald1 = _dense_layer(x[None], W1, a1s, a1d, ones,
                                   apply_act=False)
    out1, s1 = _sc_edge_layer(src, dst, als1, ald1, hs1, layer3=False)
    r1 = (1.0 / (s1[:_N] + 1e-16))[:, None]

    hs2, als2, ald2 = _dense_layer(out1, W2, a2s, a2d, r1, apply_act=True)
    out2, s2 = _sc_edge_layer(src, dst, als2, ald2, hs2, layer3=False)
    r2 = (1.0 / (s2[:_N] + 1e-16))[:, None]

    W3p = jnp.pad(W3, ((0, 0), (0, 8)))
    a3sp = jnp.pad(a3s, (0, 8))
    a3dp = jnp.pad(a3d, (0, 8))
    hs3, als3, ald3 = _dense_layer(out2, W3p, a3sp, a3dp, r2, apply_act=True)
    out3, s3h = _sc_edge_layer(src, dst, als3, ald3, hs3, layer3=True)
    s3 = s3h[0, :_N] + s3h[1, :_N]
    r3 = (1.0 / (s3 + 1e-16))[:, None]

    return _softmax(out3, r3)


# speedup vs baseline: 22.6686x; 2.3758x over previous
import functools

import jax
import jax.numpy as jnp
from jax import lax
from jax.experimental import pallas as pl
from jax.experimental.pallas import tpu as pltpu
from jax.experimental.pallas import tpu_sc as plsc

_N = 10000
_NPAD = 10240
_E = 320000
_NB = 1000
_NSUB = 16
_EB0 = _E // _NSUB


def _mm_body(x_ref, w_ref, av_ref, sc_ref, h_ref, al_ref, *, apply_act, nk):
    k = pl.program_id(2)
    x = x_ref[0]
    if apply_act:
        x = jnp.maximum(x * sc_ref[...], 0.0)
    part = jnp.dot(x, w_ref[0, 0], preferred_element_type=jnp.float32)
    if nk == 1:
        h_ref[0] = part
    else:
        @pl.when(k == 0)
        def _():
            h_ref[0] = part

        @pl.when(k != 0)
        def _():
            h_ref[0] = h_ref[0] + part

    alp = jnp.dot(part, av_ref[0], preferred_element_type=jnp.float32)
    j = pl.program_id(1)

    @pl.when((k == 0) & (j == 0))
    def _():
        al_ref[...] = alp

    @pl.when((k != 0) | (j != 0))
    def _():
        al_ref[...] = al_ref[...] + alp


def _dense_layer(xs, W, a_s, a_d, scale, apply_act):
    nk, n, kc = xs.shape
    dout = W.shape[1]
    dc = 48 if dout == 48 else 128
    nj = dout // dc
    av = jnp.stack([a_s, a_d], axis=1)
    Wr = W.reshape(nk, kc, nj, dc).transpose(0, 2, 1, 3)
    avr = av.reshape(nj, dc, 2)
    hs, al = pl.pallas_call(
        functools.partial(_mm_body, apply_act=apply_act, nk=nk),
        grid=(n // _NB, nj, nk),
        in_specs=[
            pl.BlockSpec((1, _NB, kc), lambda i, j, k: (k, i, 0)),
            pl.BlockSpec((1, 1, kc, dc), lambda i, j, k: (k, j, 0, 0)),
            pl.BlockSpec((1, dc, 2), lambda i, j, k: (j, 0, 0)),
            pl.BlockSpec((_NB, 1), lambda i, j, k: (i, 0)),
        ],
        out_specs=[
            pl.BlockSpec((1, _NB, dc), lambda i, j, k: (j, i, 0)),
            pl.BlockSpec((_NB, 2), lambda i, j, k: (i, 0)),
        ],
        out_shape=[
            jax.ShapeDtypeStruct((nj, n, dc), jnp.float32),
            jax.ShapeDtypeStruct((n, 2), jnp.float32),
        ],
    )(xs, Wr, avr, scale)
    return hs, al[:, 0], al[:, 1]


def _zero_vec(ref, rows, width):
    z = jnp.zeros((16,), jnp.float32)

    def body(i, _):
        r = i // (width // 16)
        cc = i % (width // 16)
        ref[r, pl.ds(cc * 16, 16)] = z
        return 0

    lax.fori_loop(0, rows * (width // 16), body, 0)


def _zero_vec1d(ref, size):
    z = jnp.zeros((16,), jnp.float32)

    def body(i, _):
        ref[pl.ds(i * 16, 16)] = z
        return 0

    lax.fori_loop(0, size // 16, body, 0)


def _table_max(ref, n):
    def body(i, acc):
        return jnp.maximum(acc, ref[pl.ds(i * 16, 16)])

    acc = lax.fori_loop(0, n // 16, body, jnp.full((16,), -1e30, jnp.float32))
    m = acc[0]
    for j in range(1, 16):
        m = jnp.maximum(m, acc[j])
    return m


_S = 2000
_K = 80
_NBS = _S // _K


def _phase_b_seg(src_s, dst_s, p_s, als_v, ald_v, M):
    def body(i, _):
        q = i // (_K // 16)
        m = i % (_K // 16)
        sv = src_s[pl.ds(i * 16, 16)]
        dv = dst_s[q, pl.ds(m * 16, 16)]
        av = plsc.load_gather(als_v, [sv])
        bv = plsc.load_gather(ald_v, [dv])
        ev = av + bv
        ev = jnp.where(ev >= 0.0, ev, ev * 0.2)
        p_s[q, pl.ds(m * 16, 16)] = jnp.exp(ev - M)
        return 0

    lax.fori_loop(0, _S // 16, body, 0)


def _scatter_s(p_s, dst_s, s_sh, sem, drain_src):
    def body(r, _):
        pltpu.async_copy(p_s.at[r], s_sh.at[dst_s.at[r]], sem, add=True)
        return 0

    lax.fori_loop(0, _NBS, body, 0)
    pltpu.make_async_copy(drain_src, dst_s, sem).wait()


def _zero_acc(acc, zbuf, t):
    def zb(j, _):
        pltpu.sync_copy(zbuf, acc.at[pl.ds(t * 640 + j * 80, 80)])
        return 0

    @pl.when(t < 15)
    def _():
        lax.fori_loop(0, 8, zb, 0)

    @pl.when(t == 15)
    def _():
        lax.fori_loop(0, 5, zb, 0)


def _flush_acc(acc, out_slab_hbm, t):
    @pl.when(t < 15)
    def _():
        pltpu.sync_copy(acc.at[pl.ds(t * 640, 640)],
                        out_slab_hbm.at[pl.ds(t * 640, 640)])

    @pl.when(t == 15)
    def _():
        pltpu.sync_copy(acc.at[pl.ds(9600, 400)],
                        out_slab_hbm.at[pl.ds(9600, 400)])


def _phase_c_seg(slab, src_s, dst_s, p_s, hsf_hbm, acc,
                 ib0, ib1, buf0, buf1, g0, g1, s0, s1, Dc):
    kc = _K // 16
    ncc = Dc // 16
    nb = _NBS

    def stage(b, ib, sem, buf):
        off = slab * _N

        def ibody(i, _):
            ib[pl.ds(i * 16, 16)] = src_s[pl.ds(b * _K + i * 16, 16)] + off
            return 0

        lax.fori_loop(0, kc, ibody, 0)
        pltpu.async_copy(hsf_hbm.at[ib], buf, sem)

    def wait_dma(buf, sem):
        pltpu.make_async_copy(hsf_hbm.at[pl.ds(0, _K)], buf, sem).wait()

    def multiply(b, buf):
        def mbody(g, _):
            pvec = p_s[b, pl.ds(g * 16, 16)]
            for j in range(16):
                pe = pvec[j]
                r = g * 16 + j
                for cc in range(ncc):
                    sl = pl.ds(cc * 16, 16)
                    buf[r, sl] = buf[r, sl] * pe
            return 0

        lax.fori_loop(0, _K // 16, mbody, 0)

    stage(0, ib0, g0, buf0)

    def pair(pr, _):
        b0 = 2 * pr
        wait_dma(buf0, g0)

        @pl.when(pr > 0)
        def _():
            wait_dma(buf1, s1)

        stage(b0 + 1, ib1, g1, buf1)
        multiply(b0, buf0)
        pltpu.async_copy(buf0, acc.at[dst_s.at[b0]], s0, add=True)
        wait_dma(buf1, g1)
        wait_dma(buf0, s0)

        @pl.when(b0 + 2 < nb)
        def _():
            stage(b0 + 2, ib0, g0, buf0)

        multiply(b0 + 1, buf1)
        pltpu.async_copy(buf1, acc.at[dst_s.at[b0 + 1]], s1, add=True)
        return 0

    lax.fori_loop(0, nb // 2, pair, 0)
    if nb % 2 == 1:
        wait_dma(buf0, g0)
        multiply(nb - 1, buf0)
        pltpu.async_copy(buf0, acc.at[dst_s.at[nb - 1]], s0, add=True)
        wait_dma(buf0, s0)
    wait_dma(buf1, s1)


def _sc_body(src_hbm, dst4_hbm, als_hbm, ald_hbm, hsf_hbm, out_hbm, s_hbm,
             src_s, dst_s, p_s, als_v, ald_v, ib0, ib1, buf0, buf1, zs,
             acc, s_sh, g0, g1, s0, s1, *, Dc, spc, nseg, layer3):
    c = lax.axis_index("c")
    t = lax.axis_index("s")
    grp = t * 2 + c if layer3 else t
    ebase = grp * (nseg * _S)
    pltpu.sync_copy(als_hbm, als_v)
    pltpu.sync_copy(ald_hbm, ald_v)
    _zero_vec1d(zs, 640)
    pltpu.sync_copy(zs, s_sh.at[pl.ds(t * 640, 640)])
    plsc.subcore_barrier()
    M = jnp.maximum(_table_max(als_v, _N) + _table_max(ald_v, _N), 0.0)

    for sp in range(spc):
        slab = 0 if layer3 else c * spc + sp
        _zero_vec(buf0, 80, Dc)
        _zero_acc(acc, buf0, t)
        plsc.subcore_barrier()

        def seg_body(seg, _, sp=sp):
            e0 = ebase + seg * _S
            pltpu.sync_copy(src_hbm.at[pl.ds(e0, _S)], src_s)
            pltpu.sync_copy(dst4_hbm.at[grp].at[seg], dst_s)
            _phase_b_seg(src_s, dst_s, p_s, als_v, ald_v, M)
            if sp == 0:
                _scatter_s(p_s, dst_s, s_sh, s0, dst4_hbm.at[grp].at[seg])
            _phase_c_seg(slab, src_s, dst_s, p_s, hsf_hbm, acc,
                         ib0, ib1, buf0, buf1, g0, g1, s0, s1, Dc)
            return 0

        lax.fori_loop(0, nseg, seg_body, 0)
        plsc.subcore_barrier()
        _flush_acc(acc, out_hbm.at[c] if layer3 else out_hbm.at[slab], t)

    if layer3:
        pltpu.sync_copy(s_sh.at[pl.ds(t * 640, 640)],
                        s_hbm.at[c].at[pl.ds(t * 640, 640)])
    else:
        @pl.when(c == 0)
        def _():
            pltpu.sync_copy(s_sh.at[pl.ds(t * 640, 640)],
                            s_hbm.at[pl.ds(t * 640, 640)])


def _sc_edge_layer(src, dst, als, ald, hs, *, layer3):
    nj, n, dc = hs.shape
    hsf = hs.reshape(nj * n, dc)
    mesh = plsc.VectorSubcoreMesh(core_axis_name="c", subcore_axis_name="s")
    if layer3:
        ngrp, nseg, spc = 32, 5, 1
        out_shape = jax.ShapeDtypeStruct((2, _N, dc), jnp.float32)
        s_shape = jax.ShapeDtypeStruct((2, _NPAD), jnp.float32)
    else:
        ngrp, nseg, spc = 16, 10, nj // 2
        out_shape = jax.ShapeDtypeStruct((nj, _N, dc), jnp.float32)
        s_shape = jax.ShapeDtypeStruct((_NPAD,), jnp.float32)
    body = functools.partial(_sc_body, Dc=dc, spc=spc, nseg=nseg,
                             layer3=layer3)
    dst4 = dst.reshape(ngrp, nseg, _NBS, _K)
    kfn = pl.kernel(
        body,
        out_type=[out_shape, s_shape],
        mesh=mesh,
        compiler_params=pltpu.CompilerParams(needs_layout_passes=False,
                                             use_tc_tiling_on_sc=False),
        scratch_types=[
            pltpu.VMEM((_S,), jnp.int32),
            pltpu.VMEM((_NBS, _K), jnp.int32),
            pltpu.VMEM((_NBS, _K), jnp.float32),
            pltpu.VMEM((_N,), jnp.float32),
            pltpu.VMEM((_N,), jnp.float32),
            pltpu.VMEM((_K,), jnp.int32),
            pltpu.VMEM((_K,), jnp.int32),
            pltpu.VMEM((_K, dc), jnp.float32),
            pltpu.VMEM((_K, dc), jnp.float32),
            pltpu.VMEM((640,), jnp.float32),
            pltpu.VMEM_SHARED((_N, dc), jnp.float32),
            pltpu.VMEM_SHARED((_NPAD,), jnp.float32),
            pltpu.SemaphoreType.DMA,
            pltpu.SemaphoreType.DMA,
            pltpu.SemaphoreType.DMA,
            pltpu.SemaphoreType.DMA,
        ],
    )
    return kfn(src, dst4, als, ald, hsf)


def _softmax_body(xa_ref, xb_ref, r_ref, o_ref):
    x = (xa_ref[0] + xb_ref[0]) * r_ref[...]
    col = lax.broadcasted_iota(jnp.int32, x.shape, 1)
    x = jnp.where(col < 40, x, -1e30)
    m = jnp.max(x, axis=1, keepdims=True)
    p = jnp.exp(x - m)
    o_ref[...] = (p / jnp.sum(p, axis=1, keepdims=True))[:, :40]


def _softmax(out3, r3):
    return pl.pallas_call(
        _softmax_body,
        grid=(_N // _NB,),
        in_specs=[
            pl.BlockSpec((1, _NB, 48), lambda i: (0, i, 0)),
            pl.BlockSpec((1, _NB, 48), lambda i: (1, i, 0)),
            pl.BlockSpec((_NB, 1), lambda i: (i, 0)),
        ],
        out_specs=pl.BlockSpec((_NB, 40), lambda i: (i, 0)),
        out_shape=jax.ShapeDtypeStruct((_N, 40), jnp.float32),
    )(out3, out3, r3)


def kernel(x, edge_idx, W1, a1s, a1d, W2, a2s, a2d, W3, a3s, a3d):
    src = edge_idx[0].astype(jnp.int32)
    dst = edge_idx[1].astype(jnp.int32)
    ones = jnp.ones((_N, 1), jnp.float32)

    hs1, als1, ald1 = _dense_layer(x[None], W1, a1s, a1d, ones,
                                   apply_act=False)
    out1, s1 = _sc_edge_layer(src, dst, als1, ald1, hs1, layer3=False)
    r1 = (1.0 / (s1[:_N] + 1e-16))[:, None]

    hs2, als2, ald2 = _dense_layer(out1, W2, a2s, a2d, r1, apply_act=True)
    out2, s2 = _sc_edge_layer(src, dst, als2, ald2, hs2, layer3=False)
    r2 = (1.0 / (s2[:_N] + 1e-16))[:, None]

    W3p = jnp.pad(W3, ((0, 0), (0, 8)))
    a3sp = jnp.pad(a3s, (0, 8))
    a3dp = jnp.pad(a3d, (0, 8))
    hs3, als3, ald3 = _dense_layer(out2, W3p, a3sp, a3dp, r2, apply_act=True)
    out3, s3h = _sc_edge_layer(src, dst, als3, ald3, hs3, layer3=True)
    s3 = s3h[0, :_N] + s3h[1, :_N]
    r3 = (1.0 / (s3 + 1e-16))[:, None]

    return _softmax(out3, r3)


# speedup vs baseline: 23.4831x; 1.0359x over previous
import functools

import jax
import jax.numpy as jnp
from jax import lax
from jax.experimental import pallas as pl
from jax.experimental.pallas import tpu as pltpu
from jax.experimental.pallas import tpu_sc as plsc

_N = 10000
_NPAD = 10240
_E = 320000
_NB = 1000
_NSUB = 16
_EB0 = _E // _NSUB


def _mm_body(x_ref, w_ref, av_ref, sc_ref, h_ref, al_ref, *, apply_act, nk):
    k = pl.program_id(2)
    x = x_ref[0]
    if apply_act:
        x = jnp.maximum(x * sc_ref[...], 0.0)
    part = jnp.dot(x, w_ref[0, 0], preferred_element_type=jnp.float32)
    if nk == 1:
        h_ref[0] = part
    else:
        @pl.when(k == 0)
        def _():
            h_ref[0] = part

        @pl.when(k != 0)
        def _():
            h_ref[0] = h_ref[0] + part

    alp = jnp.dot(part, av_ref[0], preferred_element_type=jnp.float32)
    j = pl.program_id(1)

    @pl.when((k == 0) & (j == 0))
    def _():
        al_ref[...] = alp

    @pl.when((k != 0) | (j != 0))
    def _():
        al_ref[...] = al_ref[...] + alp


def _dense_layer(xs, W, a_s, a_d, scale, apply_act):
    nk, n, kc = xs.shape
    dout = W.shape[1]
    dc = 48 if dout == 48 else 128
    nj = dout // dc
    av = jnp.stack([a_s, a_d], axis=1)
    Wr = W.reshape(nk, kc, nj, dc).transpose(0, 2, 1, 3)
    avr = av.reshape(nj, dc, 2)
    hs, al = pl.pallas_call(
        functools.partial(_mm_body, apply_act=apply_act, nk=nk),
        grid=(n // _NB, nj, nk),
        in_specs=[
            pl.BlockSpec((1, _NB, kc), lambda i, j, k: (k, i, 0)),
            pl.BlockSpec((1, 1, kc, dc), lambda i, j, k: (k, j, 0, 0)),
            pl.BlockSpec((1, dc, 2), lambda i, j, k: (j, 0, 0)),
            pl.BlockSpec((_NB, 1), lambda i, j, k: (i, 0)),
        ],
        out_specs=[
            pl.BlockSpec((1, _NB, dc), lambda i, j, k: (j, i, 0)),
            pl.BlockSpec((_NB, 2), lambda i, j, k: (i, 0)),
        ],
        out_shape=[
            jax.ShapeDtypeStruct((nj, n, dc), jnp.float32),
            jax.ShapeDtypeStruct((n, 2), jnp.float32),
        ],
    )(xs, Wr, avr, scale)
    return hs, al[:, 0], al[:, 1]


def _zero_vec(ref, rows, width):
    z = jnp.zeros((16,), jnp.float32)

    def body(i, _):
        r = i // (width // 16)
        cc = i % (width // 16)
        ref[r, pl.ds(cc * 16, 16)] = z
        return 0

    lax.fori_loop(0, rows * (width // 16), body, 0)


def _zero_vec1d(ref, size):
    z = jnp.zeros((16,), jnp.float32)

    def body(i, _):
        ref[pl.ds(i * 16, 16)] = z
        return 0

    lax.fori_loop(0, size // 16, body, 0)


def _table_max(ref, n):
    def body(i, acc):
        return jnp.maximum(acc, ref[pl.ds(i * 16, 16)])

    acc = lax.fori_loop(0, n // 16, body, jnp.full((16,), -1e30, jnp.float32))
    m = acc[0]
    for j in range(1, 16):
        m = jnp.maximum(m, acc[j])
    return m


_S = 2000
_K = 80
_NBS = _S // _K


def _phase_b_seg(src_s, dst_s, p_s, als_v, ald_v, M, K):
    def body(i, _):
        q = i // (K // 16)
        m = i % (K // 16)
        sv = src_s[pl.ds(i * 16, 16)]
        dv = dst_s[q, pl.ds(m * 16, 16)]
        av = plsc.load_gather(als_v, [sv])
        bv = plsc.load_gather(ald_v, [dv])
        ev = av + bv
        ev = jnp.where(ev >= 0.0, ev, ev * 0.2)
        p_s[q, pl.ds(m * 16, 16)] = jnp.exp(ev - M)
        return 0

    lax.fori_loop(0, _S // 16, body, 0)


def _scatter_s(p_s, dst_s, s_sh, sem, drain_src):
    def body(r, _):
        pltpu.async_copy(p_s.at[r], s_sh.at[dst_s.at[r]], sem, add=True)
        return 0

    lax.fori_loop(0, p_s.shape[0], body, 0)
    pltpu.make_async_copy(drain_src, dst_s, sem).wait()


def _zero_acc(acc, zbuf, t):
    z = zbuf.at[pl.ds(0, 80)]

    def zb(j, _):
        pltpu.sync_copy(z, acc.at[pl.ds(t * 640 + j * 80, 80)])
        return 0

    @pl.when(t < 15)
    def _():
        lax.fori_loop(0, 8, zb, 0)

    @pl.when(t == 15)
    def _():
        lax.fori_loop(0, 5, zb, 0)


def _flush_acc(acc, out_slab_hbm, t):
    @pl.when(t < 15)
    def _():
        pltpu.sync_copy(acc.at[pl.ds(t * 640, 640)],
                        out_slab_hbm.at[pl.ds(t * 640, 640)])

    @pl.when(t == 15)
    def _():
        pltpu.sync_copy(acc.at[pl.ds(9600, 400)],
                        out_slab_hbm.at[pl.ds(9600, 400)])


def _phase_c_seg(slab, src_s, dst_s, p_s, hsf_hbm, acc,
                 ib0, ib1, buf0, buf1, g0, g1, s0, s1, Dc, K):
    kc = K // 16
    ncc = Dc // 16
    nb = _S // K

    def stage(b, ib, sem, buf):
        off = slab * _N

        def ibody(i, _):
            ib[pl.ds(i * 16, 16)] = src_s[pl.ds(b * K + i * 16, 16)] + off
            return 0

        lax.fori_loop(0, kc, ibody, 0)
        pltpu.async_copy(hsf_hbm.at[ib], buf, sem)

    def wait_dma(buf, sem):
        pltpu.make_async_copy(hsf_hbm.at[pl.ds(0, K)], buf, sem).wait()

    def multiply(b, buf):
        def mbody(g, _):
            pvec = p_s[b, pl.ds(g * 16, 16)]
            for j in range(16):
                pe = pvec[j]
                r = g * 16 + j
                for cc in range(ncc):
                    sl = pl.ds(cc * 16, 16)
                    buf[r, sl] = buf[r, sl] * pe
            return 0

        lax.fori_loop(0, K // 16, mbody, 0)

    stage(0, ib0, g0, buf0)

    def pair(pr, _):
        b0 = 2 * pr
        wait_dma(buf0, g0)

        @pl.when(pr > 0)
        def _():
            wait_dma(buf1, s1)

        stage(b0 + 1, ib1, g1, buf1)
        multiply(b0, buf0)
        pltpu.async_copy(buf0, acc.at[dst_s.at[b0]], s0, add=True)
        wait_dma(buf1, g1)
        wait_dma(buf0, s0)

        @pl.when(b0 + 2 < nb)
        def _():
            stage(b0 + 2, ib0, g0, buf0)

        multiply(b0 + 1, buf1)
        pltpu.async_copy(buf1, acc.at[dst_s.at[b0 + 1]], s1, add=True)
        return 0

    lax.fori_loop(0, nb // 2, pair, 0)
    if nb % 2 == 1:
        wait_dma(buf0, g0)
        multiply(nb - 1, buf0)
        pltpu.async_copy(buf0, acc.at[dst_s.at[nb - 1]], s0, add=True)
        wait_dma(buf0, s0)
    wait_dma(buf1, s1)


def _sc_body(src_hbm, dst4_hbm, als_hbm, ald_hbm, hsf_hbm, out_hbm, s_hbm,
             src_s, dst_s, p_s, als_v, ald_v, ib0, ib1, buf0, buf1, zs,
             acc, s_sh, g0, g1, s0, s1, *, Dc, spc, nseg, layer3, K):
    c = lax.axis_index("c")
    t = lax.axis_index("s")
    grp = t * 2 + c if layer3 else t
    ebase = grp * (nseg * _S)
    pltpu.sync_copy(als_hbm, als_v)
    pltpu.sync_copy(ald_hbm, ald_v)
    _zero_vec1d(zs, 640)
    pltpu.sync_copy(zs, s_sh.at[pl.ds(t * 640, 640)])
    plsc.subcore_barrier()
    M = jnp.maximum(_table_max(als_v, _N) + _table_max(ald_v, _N), 0.0)

    for sp in range(spc):
        slab = 0 if layer3 else c * spc + sp
        _zero_vec(buf0, 80, Dc)
        _zero_acc(acc, buf0, t)
        plsc.subcore_barrier()

        def seg_body(seg, _, sp=sp):
            e0 = ebase + seg * _S
            pltpu.sync_copy(src_hbm.at[pl.ds(e0, _S)], src_s)
            pltpu.sync_copy(dst4_hbm.at[grp].at[seg], dst_s)
            _phase_b_seg(src_s, dst_s, p_s, als_v, ald_v, M, K)
            if sp == 0:
                _scatter_s(p_s, dst_s, s_sh, s0, dst4_hbm.at[grp].at[seg])
            _phase_c_seg(slab, src_s, dst_s, p_s, hsf_hbm, acc,
                         ib0, ib1, buf0, buf1, g0, g1, s0, s1, Dc, K)
            return 0

        lax.fori_loop(0, nseg, seg_body, 0)
        plsc.subcore_barrier()
        _flush_acc(acc, out_hbm.at[c] if layer3 else out_hbm.at[slab], t)

    if layer3:
        pltpu.sync_copy(s_sh.at[pl.ds(t * 640, 640)],
                        s_hbm.at[c].at[pl.ds(t * 640, 640)])
    else:
        @pl.when(c == 0)
        def _():
            pltpu.sync_copy(s_sh.at[pl.ds(t * 640, 640)],
                            s_hbm.at[pl.ds(t * 640, 640)])


def _sc_edge_layer(src, dst, als, ald, hs, *, layer3):
    nj, n, dc = hs.shape
    hsf = hs.reshape(nj * n, dc)
    mesh = plsc.VectorSubcoreMesh(core_axis_name="c", subcore_axis_name="s")
    if layer3:
        ngrp, nseg, spc = 32, 5, 1
        K = 400
        out_shape = jax.ShapeDtypeStruct((2, _N, dc), jnp.float32)
        s_shape = jax.ShapeDtypeStruct((2, _NPAD), jnp.float32)
    else:
        ngrp, nseg, spc = 16, 10, nj // 2
        K = _K
        out_shape = jax.ShapeDtypeStruct((nj, _N, dc), jnp.float32)
        s_shape = jax.ShapeDtypeStruct((_NPAD,), jnp.float32)
    body = functools.partial(_sc_body, Dc=dc, spc=spc, nseg=nseg,
                             layer3=layer3, K=K)
    dst4 = dst.reshape(ngrp, nseg, _S // K, K)
    kfn = pl.kernel(
        body,
        out_type=[out_shape, s_shape],
        mesh=mesh,
        compiler_params=pltpu.CompilerParams(needs_layout_passes=False,
                                             use_tc_tiling_on_sc=False),
        scratch_types=[
            pltpu.VMEM((_S,), jnp.int32),
            pltpu.VMEM((_S // K, K), jnp.int32),
            pltpu.VMEM((_S // K, K), jnp.float32),
            pltpu.VMEM((_N,), jnp.float32),
            pltpu.VMEM((_N,), jnp.float32),
            pltpu.VMEM((K,), jnp.int32),
            pltpu.VMEM((K,), jnp.int32),
            pltpu.VMEM((K, dc), jnp.float32),
            pltpu.VMEM((K, dc), jnp.float32),
            pltpu.VMEM((640,), jnp.float32),
            pltpu.VMEM_SHARED((_N, dc), jnp.float32),
            pltpu.VMEM_SHARED((_NPAD,), jnp.float32),
            pltpu.SemaphoreType.DMA,
            pltpu.SemaphoreType.DMA,
            pltpu.SemaphoreType.DMA,
            pltpu.SemaphoreType.DMA,
        ],
    )
    return kfn(src, dst4, als, ald, hsf)


def _softmax_body(xa_ref, xb_ref, r_ref, o_ref):
    x = (xa_ref[0] + xb_ref[0]) * r_ref[...]
    col = lax.broadcasted_iota(jnp.int32, x.shape, 1)
    x = jnp.where(col < 40, x, -1e30)
    m = jnp.max(x, axis=1, keepdims=True)
    p = jnp.exp(x - m)
    o_ref[...] = (p / jnp.sum(p, axis=1, keepdims=True))[:, :40]


def _softmax(out3, r3):
    return pl.pallas_call(
        _softmax_body,
        grid=(_N // _NB,),
        in_specs=[
            pl.BlockSpec((1, _NB, 48), lambda i: (0, i, 0)),
            pl.BlockSpec((1, _NB, 48), lambda i: (1, i, 0)),
            pl.BlockSpec((_NB, 1), lambda i: (i, 0)),
        ],
        out_specs=pl.BlockSpec((_NB, 40), lambda i: (i, 0)),
        out_shape=jax.ShapeDtypeStruct((_N, 40), jnp.float32),
    )(out3, out3, r3)


def kernel(x, edge_idx, W1, a1s, a1d, W2, a2s, a2d, W3, a3s, a3d):
    src = edge_idx[0].astype(jnp.int32)
    dst = edge_idx[1].astype(jnp.int32)
    ones = jnp.ones((_N, 1), jnp.float32)

    hs1, als1, ald1 = _dense_layer(x[None], W1, a1s, a1d, ones,
                                   apply_act=False)
    out1, s1 = _sc_edge_layer(src, dst, als1, ald1, hs1, layer3=False)
    r1 = (1.0 / (s1[:_N] + 1e-16))[:, None]

    hs2, als2, ald2 = _dense_layer(out1, W2, a2s, a2d, r1, apply_act=True)
    out2, s2 = _sc_edge_layer(src, dst, als2, ald2, hs2, layer3=False)
    r2 = (1.0 / (s2[:_N] + 1e-16))[:, None]

    W3p = jnp.pad(W3, ((0, 0), (0, 8)))
    a3sp = jnp.pad(a3s, (0, 8))
    a3dp = jnp.pad(a3d, (0, 8))
    hs3, als3, ald3 = _dense_layer(out2, W3p, a3sp, a3dp, r2, apply_act=True)
    out3, s3h = _sc_edge_layer(src, dst, als3, ald3, hs3, layer3=True)
    s3 = s3h[0, :_N] + s3h[1, :_N]
    r3 = (1.0 / (s3 + 1e-16))[:, None]

    return _softmax(out3, r3)


# speedup vs baseline: 24.9153x; 1.0610x over previous
import functools

import jax
import jax.numpy as jnp
from jax import lax
from jax.experimental import pallas as pl
from jax.experimental.pallas import tpu as pltpu
from jax.experimental.pallas import tpu_sc as plsc

_N = 10000
_NPAD = 10240
_E = 320000
_NB = 1000
_NSUB = 16
_EB0 = _E // _NSUB


def _mm_body(x_ref, w_ref, av_ref, sc_ref, h_ref, al_ref, *, apply_act, nk):
    k = pl.program_id(2)
    x = x_ref[0]
    if apply_act:
        x = jnp.maximum(x * sc_ref[...], 0.0)
    part = jnp.dot(x, w_ref[0, 0], preferred_element_type=jnp.float32)
    if nk == 1:
        h_ref[0] = part
    else:
        @pl.when(k == 0)
        def _():
            h_ref[0] = part

        @pl.when(k != 0)
        def _():
            h_ref[0] = h_ref[0] + part

    alp = jnp.dot(part, av_ref[0], preferred_element_type=jnp.float32)
    j = pl.program_id(1)

    @pl.when((k == 0) & (j == 0))
    def _():
        al_ref[...] = alp

    @pl.when((k != 0) | (j != 0))
    def _():
        al_ref[...] = al_ref[...] + alp


def _dense_layer(xs, W, a_s, a_d, scale, apply_act):
    nk, n, kc = xs.shape
    dout = W.shape[1]
    dc = 48 if dout == 48 else 128
    nj = dout // dc
    av = jnp.stack([a_s, a_d], axis=1)
    Wr = W.reshape(nk, kc, nj, dc).transpose(0, 2, 1, 3)
    avr = av.reshape(nj, dc, 2)
    hs, al = pl.pallas_call(
        functools.partial(_mm_body, apply_act=apply_act, nk=nk),
        grid=(n // _NB, nj, nk),
        in_specs=[
            pl.BlockSpec((1, _NB, kc), lambda i, j, k: (k, i, 0)),
            pl.BlockSpec((1, 1, kc, dc), lambda i, j, k: (k, j, 0, 0)),
            pl.BlockSpec((1, dc, 2), lambda i, j, k: (j, 0, 0)),
            pl.BlockSpec((_NB, 1), lambda i, j, k: (i, 0)),
        ],
        out_specs=[
            pl.BlockSpec((1, _NB, dc), lambda i, j, k: (j, i, 0)),
            pl.BlockSpec((_NB, 2), lambda i, j, k: (i, 0)),
        ],
        out_shape=[
            jax.ShapeDtypeStruct((nj, n, dc), jnp.float32),
            jax.ShapeDtypeStruct((n, 2), jnp.float32),
        ],
    )(xs, Wr, avr, scale)
    return hs, al[:, 0], al[:, 1]


def _zero_vec(ref, rows, width):
    z = jnp.zeros((16,), jnp.float32)

    def body(i, _):
        r = i // (width // 16)
        cc = i % (width // 16)
        ref[r, pl.ds(cc * 16, 16)] = z
        return 0

    lax.fori_loop(0, rows * (width // 16), body, 0)


def _zero_vec1d(ref, size):
    z = jnp.zeros((16,), jnp.float32)

    def body(i, _):
        ref[pl.ds(i * 16, 16)] = z
        return 0

    lax.fori_loop(0, size // 16, body, 0)


def _table_max(ref, n):
    def body(i, acc):
        return jnp.maximum(acc, ref[pl.ds(i * 16, 16)])

    acc = lax.fori_loop(0, n // 16, body, jnp.full((16,), -1e30, jnp.float32))
    m = acc[0]
    for j in range(1, 16):
        m = jnp.maximum(m, acc[j])
    return m


_S = 2000
_K = 80
_NBS = _S // _K


def _phase_b_seg(src_s, dst_s, p_s, als_v, ald_v, M, K):
    def body(i, _):
        q = i // (K // 16)
        m = i % (K // 16)
        sv = src_s[pl.ds(i * 16, 16)]
        dv = dst_s[q, pl.ds(m * 16, 16)]
        av = plsc.load_gather(als_v, [sv])
        bv = plsc.load_gather(ald_v, [dv])
        ev = av + bv
        ev = jnp.where(ev >= 0.0, ev, ev * 0.2)
        p_s[q, pl.ds(m * 16, 16)] = jnp.exp(ev - M)
        return 0

    lax.fori_loop(0, _S // 16, body, 0)


def _scatter_s(p_s, dst_s, s_sh, sem, drain_src):
    def body(r, _):
        pltpu.async_copy(p_s.at[r], s_sh.at[dst_s.at[r]], sem, add=True)
        return 0

    lax.fori_loop(0, p_s.shape[0], body, 0)
    pltpu.make_async_copy(drain_src, dst_s, sem).wait()


def _zero_acc(acc, zbuf, t):
    z = zbuf.at[pl.ds(0, 80)]

    def zb(j, _):
        pltpu.sync_copy(z, acc.at[pl.ds(t * 640 + j * 80, 80)])
        return 0

    @pl.when(t < 15)
    def _():
        lax.fori_loop(0, 8, zb, 0)

    @pl.when(t == 15)
    def _():
        lax.fori_loop(0, 5, zb, 0)


def _flush_acc(acc, out_slab_hbm, t):
    @pl.when(t < 15)
    def _():
        pltpu.sync_copy(acc.at[pl.ds(t * 640, 640)],
                        out_slab_hbm.at[pl.ds(t * 640, 640)])

    @pl.when(t == 15)
    def _():
        pltpu.sync_copy(acc.at[pl.ds(9600, 400)],
                        out_slab_hbm.at[pl.ds(9600, 400)])


def _phase_c_seg(slab, src_s, dst_s, p_s, hsf_hbm, acc,
                 ib0, ib1, buf0, buf1, g0, g0b, g1, g1b, s0, s1, Dc, K):
    kc = K // 16
    ncc = Dc // 16
    nb = _S // K
    H = K // 2

    def stage(b, ib, sem, semb, buf):
        off = slab * _N

        def ibody(i, _):
            ib[pl.ds(i * 16, 16)] = src_s[pl.ds(b * K + i * 16, 16)] + off
            return 0

        lax.fori_loop(0, kc, ibody, 0)
        pltpu.async_copy(hsf_hbm.at[ib.at[pl.ds(0, H)]],
                         buf.at[pl.ds(0, H)], sem)
        pltpu.async_copy(hsf_hbm.at[ib.at[pl.ds(H, H)]],
                         buf.at[pl.ds(H, H)], semb)

    def wait_gather(buf, sem, semb):
        pltpu.make_async_copy(hsf_hbm.at[pl.ds(0, H)],
                              buf.at[pl.ds(0, H)], sem).wait()
        pltpu.make_async_copy(hsf_hbm.at[pl.ds(0, H)],
                              buf.at[pl.ds(H, H)], semb).wait()

    def wait_dma(buf, sem):
        pltpu.make_async_copy(hsf_hbm.at[pl.ds(0, K)], buf, sem).wait()

    def multiply(b, buf):
        def mbody(g, _):
            pvec = p_s[b, pl.ds(g * 16, 16)]
            for j in range(16):
                pe = pvec[j]
                r = g * 16 + j
                for cc in range(ncc):
                    sl = pl.ds(cc * 16, 16)
                    buf[r, sl] = buf[r, sl] * pe
            return 0

        lax.fori_loop(0, K // 16, mbody, 0)

    stage(0, ib0, g0, g0b, buf0)

    def pair(pr, _):
        b0 = 2 * pr
        wait_gather(buf0, g0, g0b)

        @pl.when(pr > 0)
        def _():
            wait_dma(buf1, s1)

        stage(b0 + 1, ib1, g1, g1b, buf1)
        multiply(b0, buf0)
        pltpu.async_copy(buf0, acc.at[dst_s.at[b0]], s0, add=True)
        wait_gather(buf1, g1, g1b)
        wait_dma(buf0, s0)

        @pl.when(b0 + 2 < nb)
        def _():
            stage(b0 + 2, ib0, g0, g0b, buf0)

        multiply(b0 + 1, buf1)
        pltpu.async_copy(buf1, acc.at[dst_s.at[b0 + 1]], s1, add=True)
        return 0

    lax.fori_loop(0, nb // 2, pair, 0)
    if nb % 2 == 1:
        wait_gather(buf0, g0, g0b)
        multiply(nb - 1, buf0)
        pltpu.async_copy(buf0, acc.at[dst_s.at[nb - 1]], s0, add=True)
        wait_dma(buf0, s0)
    wait_dma(buf1, s1)


def _sc_body(src_hbm, dst4_hbm, als_hbm, ald_hbm, hsf_hbm, out_hbm, s_hbm,
             src_s, dst_s, p_s, als_v, ald_v, ib0, ib1, buf0, buf1, zs,
             acc, s_sh, g0, g0b, g1, g1b, s0, s1, *, Dc, spc, nseg, layer3,
             K):
    c = lax.axis_index("c")
    t = lax.axis_index("s")
    grp = t * 2 + c if layer3 else t
    ebase = grp * (nseg * _S)
    pltpu.sync_copy(als_hbm, als_v)
    pltpu.sync_copy(ald_hbm, ald_v)
    _zero_vec1d(zs, 640)
    pltpu.sync_copy(zs, s_sh.at[pl.ds(t * 640, 640)])
    plsc.subcore_barrier()
    M = jnp.maximum(_table_max(als_v, _N) + _table_max(ald_v, _N), 0.0)

    for sp in range(spc):
        slab = 0 if layer3 else c * spc + sp
        _zero_vec(buf0, 80, Dc)
        _zero_acc(acc, buf0, t)
        plsc.subcore_barrier()

        def seg_body(seg, _, sp=sp):
            e0 = ebase + seg * _S
            pltpu.sync_copy(src_hbm.at[pl.ds(e0, _S)], src_s)
            pltpu.sync_copy(dst4_hbm.at[grp].at[seg], dst_s)
            _phase_b_seg(src_s, dst_s, p_s, als_v, ald_v, M, K)
            if sp == 0:
                _scatter_s(p_s, dst_s, s_sh, s0, dst4_hbm.at[grp].at[seg])
            _phase_c_seg(slab, src_s, dst_s, p_s, hsf_hbm, acc,
                         ib0, ib1, buf0, buf1, g0, g0b, g1, g1b, s0, s1,
                         Dc, K)
            return 0

        lax.fori_loop(0, nseg, seg_body, 0)
        plsc.subcore_barrier()
        _flush_acc(acc, out_hbm.at[c] if layer3 else out_hbm.at[slab], t)

    if layer3:
        pltpu.sync_copy(s_sh.at[pl.ds(t * 640, 640)],
                        s_hbm.at[c].at[pl.ds(t * 640, 640)])
    else:
        @pl.when(c == 0)
        def _():
            pltpu.sync_copy(s_sh.at[pl.ds(t * 640, 640)],
                            s_hbm.at[pl.ds(t * 640, 640)])


def _sc_edge_layer(src, dst, als, ald, hs, *, layer3):
    nj, n, dc = hs.shape
    hsf = hs.reshape(nj * n, dc)
    mesh = plsc.VectorSubcoreMesh(core_axis_name="c", subcore_axis_name="s")
    if layer3:
        ngrp, nseg, spc = 32, 5, 1
        K = 400
        out_shape = jax.ShapeDtypeStruct((2, _N, dc), jnp.float32)
        s_shape = jax.ShapeDtypeStruct((2, _NPAD), jnp.float32)
    else:
        ngrp, nseg, spc = 16, 10, nj // 2
        K = _K
        out_shape = jax.ShapeDtypeStruct((nj, _N, dc), jnp.float32)
        s_shape = jax.ShapeDtypeStruct((_NPAD,), jnp.float32)
    body = functools.partial(_sc_body, Dc=dc, spc=spc, nseg=nseg,
                             layer3=layer3, K=K)
    dst4 = dst.reshape(ngrp, nseg, _S // K, K)
    kfn = pl.kernel(
        body,
        out_type=[out_shape, s_shape],
        mesh=mesh,
        compiler_params=pltpu.CompilerParams(needs_layout_passes=False,
                                             use_tc_tiling_on_sc=False),
        scratch_types=[
            pltpu.VMEM((_S,), jnp.int32),
            pltpu.VMEM((_S // K, K), jnp.int32),
            pltpu.VMEM((_S // K, K), jnp.float32),
            pltpu.VMEM((_N,), jnp.float32),
            pltpu.VMEM((_N,), jnp.float32),
            pltpu.VMEM((K,), jnp.int32),
            pltpu.VMEM((K,), jnp.int32),
            pltpu.VMEM((K, dc), jnp.float32),
            pltpu.VMEM((K, dc), jnp.float32),
            pltpu.VMEM((640,), jnp.float32),
            pltpu.VMEM_SHARED((_N, dc), jnp.float32),
            pltpu.VMEM_SHARED((_NPAD,), jnp.float32),
            pltpu.SemaphoreType.DMA,
            pltpu.SemaphoreType.DMA,
            pltpu.SemaphoreType.DMA,
            pltpu.SemaphoreType.DMA,
            pltpu.SemaphoreType.DMA,
            pltpu.SemaphoreType.DMA,
        ],
    )
    return kfn(src, dst4, als, ald, hsf)


def _softmax_body(xa_ref, xb_ref, r_ref, o_ref):
    x = (xa_ref[0] + xb_ref[0]) * r_ref[...]
    col = lax.broadcasted_iota(jnp.int32, x.shape, 1)
    x = jnp.where(col < 40, x, -1e30)
    m = jnp.max(x, axis=1, keepdims=True)
    p = jnp.exp(x - m)
    o_ref[...] = (p / jnp.sum(p, axis=1, keepdims=True))[:, :40]


def _softmax(out3, r3):
    return pl.pallas_call(
        _softmax_body,
        grid=(_N // _NB,),
        in_specs=[
            pl.BlockSpec((1, _NB, 48), lambda i: (0, i, 0)),
            pl.BlockSpec((1, _NB, 48), lambda i: (1, i, 0)),
            pl.BlockSpec((_NB, 1), lambda i: (i, 0)),
        ],
        out_specs=pl.BlockSpec((_NB, 40), lambda i: (i, 0)),
        out_shape=jax.ShapeDtypeStruct((_N, 40), jnp.float32),
    )(out3, out3, r3)


def kernel(x, edge_idx, W1, a1s, a1d, W2, a2s, a2d, W3, a3s, a3d):
    src = edge_idx[0].astype(jnp.int32)
    dst = edge_idx[1].astype(jnp.int32)
    ones = jnp.ones((_N, 1), jnp.float32)

    hs1, als1, ald1 = _dense_layer(x[None], W1, a1s, a1d, ones,
                                   apply_act=False)
    out1, s1 = _sc_edge_layer(src, dst, als1, ald1, hs1, layer3=False)
    r1 = (1.0 / (s1[:_N] + 1e-16))[:, None]

    hs2, als2, ald2 = _dense_layer(out1, W2, a2s, a2d, r1, apply_act=True)
    out2, s2 = _sc_edge_layer(src, dst, als2, ald2, hs2, layer3=False)
    r2 = (1.0 / (s2[:_N] + 1e-16))[:, None]

    W3p = jnp.pad(W3, ((0, 0), (0, 8)))
    a3sp = jnp.pad(a3s, (0, 8))
    a3dp = jnp.pad(a3d, (0, 8))
    hs3, als3, ald3 = _dense_layer(out2, W3p, a3sp, a3dp, r2, apply_act=True)
    out3, s3h = _sc_edge_layer(src, dst, als3, ald3, hs3, layer3=True)
    s3 = s3h[0, :_N] + s3h[1, :_N]
    r3 = (1.0 / (s3 + 1e-16))[:, None]

    return _softmax(out3, r3)
